# baseline (speedup 1.0000x reference)
import functools

import jax
import jax.numpy as jnp
import numpy as np
from jax import lax
from jax.experimental import pallas as pl
from jax.experimental.pallas import tpu as pltpu
from jax.experimental.pallas import tpu_sc as plsc

VOCAB = 1000000
D = 64
B = 4096
L = 200
BL = B * L

NC = 2
NS = 16
NW = NC * NS
ROWS_PER_W = BL // NW
CH = 128
N_CH = ROWS_PER_W // CH

NBLK = VOCAB // CH
VTAIL = NBLK * CH
NT = NBLK // NW
NT_EXTRA = NBLK - NW * NT


def _positional_encoding_np(seq_len, d_model):
    pos = np.arange(seq_len, dtype=np.float32)[:, None]
    i = np.arange(0, d_model, 2, dtype=np.float32)[None, :]
    angles = pos / np.power(10000.0, i / d_model)
    pe = np.zeros((seq_len, d_model), dtype=np.float32)
    pe[:, 0::2] = np.sin(angles)
    pe[:, 1::2] = np.cos(angles)
    return pe


_MESH = plsc.VectorSubcoreMesh(
    core_axis_name="c", subcore_axis_name="s", num_cores=NC, num_subcores=NS
)


@functools.partial(
    pl.kernel,
    mesh=_MESH,
    out_type=jax.ShapeDtypeStruct((VOCAB, 2 * D), jnp.float32),
    scratch_types=[
        pltpu.VMEM((2, D, CH), jnp.float32),
        pltpu.VMEM((2, CH, 2 * D), jnp.float32),
        pltpu.VMEM((D, D), jnp.float32),
        pltpu.SemaphoreType.DMA,
        pltpu.SemaphoreType.DMA,
        pltpu.SemaphoreType.DMA,
        pltpu.SemaphoreType.DMA,
    ],
    compiler_params=pltpu.CompilerParams(
        use_tc_tiling_on_sc=True, needs_layout_passes=False
    ),
)
def _format_table(tt_hbm, tail_hbm, tbl_hbm, q_v, o_v, tail_v,
                  ss0, ss1, sw0, sw1):
    ss = (ss0, ss1)
    sw = (sw0, sw1)
    wid = lax.axis_index("s") * NC + lax.axis_index("c")

    def blk(t):
        return NW * t + wid

    def stage(t, b):
        start = pl.multiple_of(blk(t) * CH, CH)
        for a in range(D // 8):
            pltpu.async_copy(
                tt_hbm.at[pl.ds(8 * a, 8), pl.ds(start, CH)],
                q_v.at[b, pl.ds(8 * a, 8)],
                ss[b],
            )

    def wait_stage(b):
        for a in range(D // 8):
            pltpu.make_async_copy(
                tt_hbm.at[pl.ds(0, 8), pl.ds(0, CH)],
                q_v.at[b, pl.ds(8 * a, 8)],
                ss[b],
            ).wait()

    def wait_write(b):
        pltpu.make_async_copy(
            o_v.at[b], tbl_hbm.at[pl.ds(0, CH)], sw[b]
        ).wait()

    stage(0, 0)

    fidx = [lax.iota(jnp.int32, 16) + 16 * s for s in range(D // 16)]

    def body(t, b):
        @pl.when(NW * (t + 1) + wid < NBLK)
        def _():
            stage(t + 1, 1 - b)
        wait_stage(b)

        @plsc.parallel_loop(0, CH, unroll=8)
        def _(vc):
            vc16 = jnp.full((16,), vc, dtype=jnp.int32)
            for s in range(D // 16):
                o_v[b, vc, pl.ds(s * 16, 16)] = plsc.load_gather(
                    q_v.at[b], [fidx[s], vc16]
                )

        @pl.when(t >= 2)
        def _():
            wait_write(b)
        ostart = pl.multiple_of(blk(t) * CH, CH)
        pltpu.async_copy(o_v.at[b], tbl_hbm.at[pl.ds(ostart, CH)], sw[b])

    def step(k, carry):
        for b in range(2):
            t = 2 * k + b

            @pl.when(NW * t + wid < NBLK)
            def _():
                body(t, b)
        return carry

    lax.fori_loop(0, (NT + 2) // 2, step, 0)
    wait_write(0)
    wait_write(1)

    @pl.when(wid == 0)
    def _():
        pltpu.sync_copy(tail_hbm, tail_v)

        def tailrow(r, carry_):
            for s in range(D // 16):
                sl = pl.ds(s * 16, 16)
                o_v[0, r, sl] = tail_v[r, sl]
            return carry_

        lax.fori_loop(0, D, tailrow, 0)
        pltpu.sync_copy(
            o_v.at[0, pl.ds(0, D)], tbl_hbm.at[pl.ds(VTAIL, D)]
        )


@functools.partial(
    pl.kernel,
    mesh=_MESH,
    out_type=jax.ShapeDtypeStruct((BL, D), jnp.float32),
    scratch_types=[
        pltpu.VMEM((2, CH), jnp.int32),
        pltpu.VMEM((2, CH, 2 * D), jnp.float32),
        pltpu.VMEM((2, CH, D), jnp.float32),
        pltpu.VMEM((2 * L, D), jnp.float32),
        pltpu.SemaphoreType.DMA,
        pltpu.SemaphoreType.DMA,
        pltpu.SemaphoreType.DMA,
        pltpu.SemaphoreType.DMA,
    ],
    compiler_params=pltpu.CompilerParams(use_tc_tiling_on_sc=True),
)
def _seq_embed(x_hbm, pe_hbm, tbl_hbm, out_hbm, idx_v, rows_v, res_v, pe_v,
               sg0, sg1, so0, so1):
    sg = (sg0, sg1)
    so = (so0, so1)
    wid = lax.axis_index("s") * NC + lax.axis_index("c")
    base = wid * ROWS_PER_W
    pltpu.sync_copy(pe_hbm, pe_v)

    def stage_idx(c, b):
        start = pl.multiple_of(base + CH * c, CH)
        pltpu.sync_copy(x_hbm.at[pl.ds(start, CH)], idx_v.at[b])

    def gather(b):
        pltpu.async_copy(tbl_hbm.at[idx_v.at[b]], rows_v.at[b], sg[b])

    def wait_gather(b):
        pltpu.make_async_copy(
            tbl_hbm.at[pl.ds(0, CH)], rows_v.at[b], sg[b]
        ).wait()

    def wait_out(b):
        pltpu.make_async_copy(
            res_v.at[b], out_hbm.at[pl.ds(0, CH)], so[b]
        ).wait()

    stage_idx(0, 0)
    gather(0)

    def step(k, carry):
        for b in range(2):
            c = 2 * k + b
            if b == 0:
                stage_idx(c + 1, 1)
                gather(1)
            else:
                @pl.when(k <= (N_CH // 2) - 2)
                def _():
                    stage_idx(c + 1, 0)
                    gather(0)
            @pl.when(k >= 1)
            def _():
                wait_out(b)
            wait_gather(b)
            p = lax.rem(CH * c, L)

            @plsc.parallel_loop(0, CH, unroll=16)
            def _(j):
                for s in range(D // 16):
                    sl = pl.ds(s * 16, 16)
                    res_v[b, j, sl] = rows_v[b, j, sl] + pe_v[p + j, sl]
            ostart = pl.multiple_of(base + CH * c, CH)
            pltpu.async_copy(
                res_v.at[b], out_hbm.at[pl.ds(ostart, CH)], so[b]
            )
        return carry

    lax.fori_loop(0, N_CH // 2, step, 0)
    wait_out(0)
    wait_out(1)


def kernel(x, table):
    pe2 = np.concatenate([_positional_encoding_np(L, D)] * 2, axis=0)
    x_flat = x.reshape(-1).astype(jnp.int32)
    tail = table[VTAIL:]
    tbl128 = _format_table(table.T, tail)
    out = _seq_embed(x_flat, jnp.asarray(pe2), tbl128)
    return out.reshape(B, L, D)


# speedup vs baseline: 1.0864x; 1.0864x over previous
import functools

import jax
import jax.numpy as jnp
import numpy as np
from jax import lax
from jax.experimental import pallas as pl
from jax.experimental.pallas import tpu as pltpu
from jax.experimental.pallas import tpu_sc as plsc

VOCAB = 1000000
D = 64
B = 4096
L = 200
BL = B * L

NC = 2
NS = 16
NW = NC * NS
ROWS_PER_W = BL // NW
CH = 128
N_CH = ROWS_PER_W // CH


def _positional_encoding_np(seq_len, d_model):
    pos = np.arange(seq_len, dtype=np.float32)[:, None]
    i = np.arange(0, d_model, 2, dtype=np.float32)[None, :]
    angles = pos / np.power(10000.0, i / d_model)
    pe = np.zeros((seq_len, d_model), dtype=np.float32)
    pe[:, 0::2] = np.sin(angles)
    pe[:, 1::2] = np.cos(angles)
    return pe


_MESH = plsc.VectorSubcoreMesh(
    core_axis_name="c", subcore_axis_name="s", num_cores=NC, num_subcores=NS
)


@functools.partial(
    pl.kernel,
    mesh=_MESH,
    out_type=jax.ShapeDtypeStruct((BL, D), jnp.float32),
    scratch_types=[
        pltpu.VMEM((2, CH), jnp.int32),
        pltpu.VMEM((2, CH), jnp.int32),
        pltpu.VMEM((2, CH, 2 * D), jnp.float32),
        pltpu.VMEM((2, CH, D), jnp.float32),
        pltpu.VMEM((2 * L, D), jnp.float32),
        pltpu.SemaphoreType.DMA,
        pltpu.SemaphoreType.DMA,
        pltpu.SemaphoreType.DMA,
        pltpu.SemaphoreType.DMA,
    ],
    compiler_params=pltpu.CompilerParams(use_tc_tiling_on_sc=True),
)
def _seq_embed(x_hbm, pe_hbm, tbl_hbm, out_hbm, idxr_v, idx2_v, rows_v,
               res_v, pe_v, sg0, sg1, so0, so1):
    sg = (sg0, sg1)
    so = (so0, so1)
    wid = lax.axis_index("s") * NC + lax.axis_index("c")
    base = wid * ROWS_PER_W
    pltpu.sync_copy(pe_hbm, pe_v)

    def stage_idx(c, b):
        start = pl.multiple_of(base + CH * c, CH)
        pltpu.sync_copy(x_hbm.at[pl.ds(start, CH)], idxr_v.at[b])
        for s in range(CH // 16):
            sl = pl.ds(s * 16, 16)
            idx2_v[b, sl] = lax.shift_right_logical(idxr_v[b, sl], 1)

    def gather(b):
        pltpu.async_copy(tbl_hbm.at[idx2_v.at[b]], rows_v.at[b], sg[b])

    def wait_gather(b):
        pltpu.make_async_copy(
            tbl_hbm.at[pl.ds(0, CH)], rows_v.at[b], sg[b]
        ).wait()

    def wait_out(b):
        pltpu.make_async_copy(
            res_v.at[b], out_hbm.at[pl.ds(0, CH)], so[b]
        ).wait()

    stage_idx(0, 0)
    gather(0)

    def step(k, carry):
        for b in range(2):
            c = 2 * k + b
            if b == 0:
                stage_idx(c + 1, 1)
                gather(1)
            else:
                @pl.when(k <= (N_CH // 2) - 2)
                def _():
                    stage_idx(c + 1, 0)
                    gather(0)
            @pl.when(k >= 1)
            def _():
                wait_out(b)
            wait_gather(b)
            p = lax.rem(CH * c, L)

            @plsc.parallel_loop(0, CH // 16, unroll=2)
            def _(jg):
                j0 = jg * 16
                v16 = idxr_v[b, pl.ds(j0, 16)]
                h16 = lax.mul(lax.rem(v16, 2), D)
                for r in range(16):
                    j = j0 + r
                    h = h16[r]
                    for s in range(D // 16):
                        res_v[b, j, pl.ds(s * 16, 16)] = (
                            rows_v[b, j, pl.ds(h + s * 16, 16)]
                            + pe_v[p + j, pl.ds(s * 16, 16)]
                        )

            ostart = pl.multiple_of(base + CH * c, CH)
            pltpu.async_copy(
                res_v.at[b], out_hbm.at[pl.ds(ostart, CH)], so[b]
            )
        return carry

    lax.fori_loop(0, N_CH // 2, step, 0)
    wait_out(0)
    wait_out(1)


def kernel(x, table):
    pe2 = np.concatenate([_positional_encoding_np(L, D)] * 2, axis=0)
    x_flat = x.reshape(-1).astype(jnp.int32)
    tbl128 = table.reshape(VOCAB // 2, 2 * D)
    out = _seq_embed(x_flat, jnp.asarray(pe2), tbl128)
    return out.reshape(B, L, D)


# speedup vs baseline: 1.2555x; 1.1556x over previous
import functools

import jax
import jax.numpy as jnp
import numpy as np
from jax import lax
from jax.experimental import pallas as pl
from jax.experimental.pallas import tpu as pltpu
from jax.experimental.pallas import tpu_sc as plsc

VOCAB = 1000000
D = 64
B = 4096
L = 200
BL = B * L

NC = 2
NS = 16
NW = NC * NS
ROWS_PER_W = BL // NW
CH = 128
N_CH = ROWS_PER_W // CH


def _positional_encoding_np(seq_len, d_model):
    pos = np.arange(seq_len, dtype=np.float32)[:, None]
    i = np.arange(0, d_model, 2, dtype=np.float32)[None, :]
    angles = pos / np.power(10000.0, i / d_model)
    pe = np.zeros((seq_len, d_model), dtype=np.float32)
    pe[:, 0::2] = np.sin(angles)
    pe[:, 1::2] = np.cos(angles)
    return pe


_MESH = plsc.VectorSubcoreMesh(
    core_axis_name="c", subcore_axis_name="s", num_cores=NC, num_subcores=NS
)


@functools.partial(
    pl.kernel,
    mesh=_MESH,
    out_type=jax.ShapeDtypeStruct((BL, D), jnp.float32),
    scratch_types=[
        pltpu.VMEM((2, CH), jnp.int32),
        pltpu.VMEM((2, CH, 2 * D), jnp.float32),
        pltpu.VMEM((2, CH, D), jnp.float32),
        pltpu.VMEM((2 * L, D), jnp.float32),
        pltpu.SemaphoreType.DMA,
        pltpu.SemaphoreType.DMA,
        pltpu.SemaphoreType.DMA,
        pltpu.SemaphoreType.DMA,
    ],
    compiler_params=pltpu.CompilerParams(use_tc_tiling_on_sc=True),
)
def _seq_embed(x_hbm, pe_hbm, tbl_hbm, out_hbm, idxr_v, rows_v,
               res_v, pe_v, sg0, sg1, so0, so1):
    sg = (sg0, sg1)
    so = (so0, so1)
    wid = lax.axis_index("s") * NC + lax.axis_index("c")
    base = wid * ROWS_PER_W
    pltpu.sync_copy(pe_hbm, pe_v)

    def stage_idx(c, b):
        start = pl.multiple_of(base + CH * c, CH)
        pltpu.sync_copy(x_hbm.at[pl.ds(start, CH)], idxr_v.at[b])

    def gather(b):
        pltpu.async_copy(tbl_hbm.at[idxr_v.at[b]], rows_v.at[b], sg[b])

    def wait_gather(b):
        pltpu.make_async_copy(
            tbl_hbm.at[pl.ds(0, CH)], rows_v.at[b], sg[b]
        ).wait()

    def wait_out(b):
        pltpu.make_async_copy(
            res_v.at[b], out_hbm.at[pl.ds(0, CH)], so[b]
        ).wait()

    stage_idx(0, 0)
    gather(0)

    def step(k, carry):
        for b in range(2):
            c = 2 * k + b
            if b == 0:
                stage_idx(c + 1, 1)
                gather(1)
            else:
                @pl.when(k <= (N_CH // 2) - 2)
                def _():
                    stage_idx(c + 1, 0)
                    gather(0)
            @pl.when(k >= 1)
            def _():
                wait_out(b)
            wait_gather(b)
            p = lax.rem(CH * c, L)

            @plsc.parallel_loop(0, CH, unroll=8)
            def _(j):
                for s in range(D // 16):
                    sl = pl.ds(s * 16, 16)
                    res_v[b, j, sl] = rows_v[b, j, sl] + pe_v[p + j, sl]

            ostart = pl.multiple_of(base + CH * c, CH)
            pltpu.async_copy(
                res_v.at[b], out_hbm.at[pl.ds(ostart, CH)], so[b]
            )
        return carry

    lax.fori_loop(0, N_CH // 2, step, 0)
    wait_out(0)
    wait_out(1)


def kernel(x, table):
    pe2 = np.concatenate([_positional_encoding_np(L, D)] * 2, axis=0)
    x_flat = x.reshape(-1).astype(jnp.int32)
    tbl128 = jnp.pad(table, ((0, 0), (0, D)))
    out = _seq_embed(x_flat, jnp.asarray(pe2), tbl128)
    return out.reshape(B, L, D)
